# Initial kernel scaffold; baseline (speedup 1.0000x reference)
import jax
import jax.numpy as jnp
from jax.experimental import pallas as pl


def kernel(input, relation_embeds, edge_list, edge_type, W1, b1, W2, b2, c_r, W, W_r, a):
    raise NotImplementedError("write your pallas kernel here")


# speedup vs baseline: 9.0388x; 9.0388x over previous
import functools

import jax
import jax.numpy as jnp
from jax import lax
from jax.experimental import pallas as pl
from jax.experimental.pallas import tpu as pltpu
from jax.experimental.pallas import tpu_sc as plsc

N = 10000
E = 320000
D = 128
RP = 128
HID = 512
TW = 144
NC, NS = 2, 16
NW = NC * NS
EPW = E // NW
C = 80
NCHUNK = EPW // C
NPAD = 10240
RPT = NPAD // NS
ALPHA = 0.2


def _tables_body(x_ref, rel_ref, w1_ref, b1_ref, w2_ref, b2_ref, cr_ref,
                 w_ref, wr_ref, a_ref, s_ref, t_ref, s3_ref):
    x = x_ref[...]
    rel = rel_ref[...]
    h1 = jnp.maximum(
        jnp.dot(rel, w1_ref[...], preferred_element_type=jnp.float32)
        + b1_ref[...], 0.0)
    theta = (jnp.dot(h1, w2_ref[...], preferred_element_type=jnp.float32)
             + b2_ref[...])
    a1 = a_ref[:, 0:D]
    a2 = a_ref[:, D:2 * D]
    a3 = a_ref[:, 2 * D:3 * D]
    nt = (((1,), (1,)), ((), ()))
    v = lax.dot_general(a2, wr_ref[...], nt,
                        preferred_element_type=jnp.float32)
    c2 = jnp.sum(cr_ref[...] * v)
    q = jnp.cos(theta) * v
    wa1 = lax.dot_general(a1, w_ref[...], nt,
                          preferred_element_type=jnp.float32)
    wa3 = lax.dot_general(a3, w_ref[...], nt,
                          preferred_element_type=jnp.float32)
    spart = lax.dot_general(x, wa1, nt,
                            preferred_element_type=jnp.float32)
    s3 = lax.dot_general(x, wa3, nt,
                         preferred_element_type=jnp.float32)
    s_ref[...] = lax.dot_general(x, q, nt,
                                 preferred_element_type=jnp.float32) + spart + c2
    t_ref[:, 0:D] = jnp.dot(x, w_ref[...], preferred_element_type=jnp.float32)
    t_ref[:, D:D + 1] = jnp.ones_like(s3)
    t_ref[:, D + 1:TW] = jnp.zeros((x.shape[0], TW - D - 1), jnp.float32)
    s3_ref[...] = s3


def _build_tables(x, relp, w1, b1, w2, b2, c_r, w, w_r, a):
    bn = 2000
    return pl.pallas_call(
        _tables_body,
        grid=(N // bn,),
        in_specs=[
            pl.BlockSpec((bn, D), lambda i: (i, 0)),
            pl.BlockSpec((RP, D), lambda i: (0, 0)),
            pl.BlockSpec((D, HID), lambda i: (0, 0)),
            pl.BlockSpec((1, HID), lambda i: (0, 0)),
            pl.BlockSpec((HID, D), lambda i: (0, 0)),
            pl.BlockSpec((1, D), lambda i: (0, 0)),
            pl.BlockSpec((1, D), lambda i: (0, 0)),
            pl.BlockSpec((D, D), lambda i: (0, 0)),
            pl.BlockSpec((D, D), lambda i: (0, 0)),
            pl.BlockSpec((1, 3 * D), lambda i: (0, 0)),
        ],
        out_specs=[
            pl.BlockSpec((bn, RP), lambda i: (i, 0)),
            pl.BlockSpec((bn, TW), lambda i: (i, 0)),
            pl.BlockSpec((bn, 1), lambda i: (i, 0)),
        ],
        out_shape=[
            jax.ShapeDtypeStruct((N, RP), jnp.float32),
            jax.ShapeDtypeStruct((N, TW), jnp.float32),
            jax.ShapeDtypeStruct((N, 1), jnp.float32),
        ],
    )(x, relp, w1, b1, w2, b2, c_r, w, w_r, a)


def _edge_body(s2_hbm, s3_hbm, t_hbm, src_hbm, dst_hbm, et_hbm, z_hbm, out_hbm,
               acc, src_v, dst_v, et_v, sidx_v, g1_v, s3_v, rows_v,
               sem1, sem2, sem3):
    cid = lax.axis_index("c")
    sid = lax.axis_index("s")
    wid = cid * NS + sid
    pltpu.sync_copy(z_hbm.at[pl.ds(sid * RPT, RPT)],
                    acc.at[pl.ds(sid * RPT, RPT)])
    plsc.subcore_barrier()

    def chunk(j, carry):
        off = wid * EPW + j * C
        pltpu.sync_copy(src_hbm.at[pl.ds(off, C)], src_v)
        pltpu.sync_copy(dst_hbm.at[pl.ds(off, C)], dst_v)
        pltpu.sync_copy(et_hbm.at[pl.ds(off, C)], et_v)
        for k in range(C // 16):
            sl = pl.ds(k * 16, 16)
            sidx_v[sl] = src_v[sl] * RP + et_v[sl]
        cp1 = pltpu.async_copy(s2_hbm.at[sidx_v], g1_v, sem1)
        cp2 = pltpu.async_copy(t_hbm.at[dst_v], rows_v, sem2)
        cp3 = pltpu.async_copy(s3_hbm.at[dst_v], s3_v, sem3)
        cp1.wait()
        cp2.wait()
        cp3.wait()
        for k in range(C // 16):
            sl = pl.ds(k * 16, 16)
            sc = g1_v[sl] + s3_v[sl]
            lr = jnp.where(sc > 0.0, sc, sc * ALPHA)
            ee = jnp.exp(-lr)
            for j in range(16):
                eei = jnp.broadcast_to(lax.slice(ee, (j,), (j + 1,)), (16,))
                i = k * 16 + j
                for cc in range(TW // 16):
                    slc = pl.ds(cc * 16, 16)
                    rows_v[i, slc] = rows_v[i, slc] * eei
        pltpu.sync_copy(rows_v, acc.at[src_v], add=True)
        return carry

    lax.fori_loop(0, NCHUNK, chunk, 0)
    plsc.subcore_barrier()
    pltpu.sync_copy(acc.at[pl.ds(sid * RPT, RPT)],
                    out_hbm.at[cid, pl.ds(sid * RPT, RPT)])


_edge_kernel = functools.partial(
    pl.kernel,
    out_type=jax.ShapeDtypeStruct((NC, NPAD, TW), jnp.float32),
    mesh=plsc.VectorSubcoreMesh(core_axis_name="c", subcore_axis_name="s"),
    compiler_params=pltpu.CompilerParams(use_tc_tiling_on_sc=False),
    scratch_types=[
        pltpu.VMEM_SHARED((NPAD, TW), jnp.float32),
        pltpu.VMEM((C,), jnp.int32),
        pltpu.VMEM((C,), jnp.int32),
        pltpu.VMEM((C,), jnp.int32),
        pltpu.VMEM((C,), jnp.int32),
        pltpu.VMEM((C,), jnp.float32),
        pltpu.VMEM((C,), jnp.float32),
        pltpu.VMEM((C, TW), jnp.float32),
        pltpu.SemaphoreType.DMA,
        pltpu.SemaphoreType.DMA,
        pltpu.SemaphoreType.DMA,
    ],
)(_edge_body)


def _combine_body(p_ref, o_ref):
    h = p_ref[0, :, 0:D] + p_ref[1, :, 0:D]
    rs = p_ref[0, :, D:D + 1] + p_ref[1, :, D:D + 1]
    den = jnp.where(rs == 0.0, 1e-12, rs)
    o = h / den
    o_ref[...] = jnp.where(o > 0.0, o, jnp.exp(jnp.minimum(o, 0.0)) - 1.0)


def _combine(partial):
    bo = 1000
    return pl.pallas_call(
        _combine_body,
        grid=(N // bo,),
        in_specs=[pl.BlockSpec((NC, bo, TW), lambda i: (0, i, 0))],
        out_specs=pl.BlockSpec((bo, D), lambda i: (i, 0)),
        out_shape=jax.ShapeDtypeStruct((N, D), jnp.float32),
    )(partial)


def kernel(input, relation_embeds, edge_list, edge_type, W1, b1, W2, b2,
           c_r, W, W_r, a):
    relp = jnp.pad(relation_embeds, ((0, RP - relation_embeds.shape[0]), (0, 0)))
    s_tab, t_tab, s3_tab = _build_tables(
        input, relp, W1, b1.reshape(1, HID), W2, b2.reshape(1, D),
        c_r, W, W_r, a)
    s2 = s_tab.reshape(N * RP)
    s3 = s3_tab.reshape(N)
    src = edge_list[0]
    dst = edge_list[1]
    zeros = jnp.zeros((NPAD, TW), jnp.float32)
    partial = _edge_kernel(s2, s3, t_tab, src, dst, edge_type, zeros)
    return _combine(partial)


# speedup vs baseline: 13.8587x; 1.5332x over previous
import functools

import jax
import jax.numpy as jnp
from jax import lax
from jax.experimental import pallas as pl
from jax.experimental.pallas import tpu as pltpu
from jax.experimental.pallas import tpu_sc as plsc

N = 10000
E = 320000
D = 128
RP = 128
HID = 512
TW = 144
NC, NS = 2, 16
NW = NC * NS
EPW = E // NW
C = 80
NCHUNK = EPW // C
NPAD = 10240
RPT = NPAD // NS
ALPHA = 0.2


def _tables_body(x_ref, rel_ref, w1_ref, b1_ref, w2_ref, b2_ref, cr_ref,
                 w_ref, wr_ref, a_ref, s_ref, t_ref, s3_ref):
    x = x_ref[...]
    rel = rel_ref[...]
    h1 = jnp.maximum(
        jnp.dot(rel, w1_ref[...], preferred_element_type=jnp.float32)
        + b1_ref[...], 0.0)
    theta = (jnp.dot(h1, w2_ref[...], preferred_element_type=jnp.float32)
             + b2_ref[...])
    a1 = a_ref[:, 0:D]
    a2 = a_ref[:, D:2 * D]
    a3 = a_ref[:, 2 * D:3 * D]
    nt = (((1,), (1,)), ((), ()))
    v = lax.dot_general(a2, wr_ref[...], nt,
                        preferred_element_type=jnp.float32)
    c2 = jnp.sum(cr_ref[...] * v)
    q = jnp.cos(theta) * v
    wa1 = lax.dot_general(a1, w_ref[...], nt,
                          preferred_element_type=jnp.float32)
    wa3 = lax.dot_general(a3, w_ref[...], nt,
                          preferred_element_type=jnp.float32)
    spart = lax.dot_general(x, wa1, nt,
                            preferred_element_type=jnp.float32)
    s3 = lax.dot_general(x, wa3, nt,
                         preferred_element_type=jnp.float32)
    s_ref[...] = lax.dot_general(x, q, nt,
                                 preferred_element_type=jnp.float32) + spart + c2
    t_ref[:, 0:D] = jnp.dot(x, w_ref[...], preferred_element_type=jnp.float32)
    t_ref[:, D:D + 1] = jnp.ones_like(s3)
    t_ref[:, D + 1:TW] = jnp.zeros((x.shape[0], TW - D - 1), jnp.float32)
    s3_ref[...] = s3


def _build_tables(x, relp, w1, b1, w2, b2, c_r, w, w_r, a):
    bn = 2000
    return pl.pallas_call(
        _tables_body,
        grid=(N // bn,),
        in_specs=[
            pl.BlockSpec((bn, D), lambda i: (i, 0)),
            pl.BlockSpec((RP, D), lambda i: (0, 0)),
            pl.BlockSpec((D, HID), lambda i: (0, 0)),
            pl.BlockSpec((1, HID), lambda i: (0, 0)),
            pl.BlockSpec((HID, D), lambda i: (0, 0)),
            pl.BlockSpec((1, D), lambda i: (0, 0)),
            pl.BlockSpec((1, D), lambda i: (0, 0)),
            pl.BlockSpec((D, D), lambda i: (0, 0)),
            pl.BlockSpec((D, D), lambda i: (0, 0)),
            pl.BlockSpec((1, 3 * D), lambda i: (0, 0)),
        ],
        out_specs=[
            pl.BlockSpec((bn, RP), lambda i: (i, 0)),
            pl.BlockSpec((bn, TW), lambda i: (i, 0)),
            pl.BlockSpec((bn, 1), lambda i: (i, 0)),
        ],
        out_shape=[
            jax.ShapeDtypeStruct((N, RP), jnp.float32),
            jax.ShapeDtypeStruct((N, TW), jnp.float32),
            jax.ShapeDtypeStruct((N, 1), jnp.float32),
        ],
    )(x, relp, w1, b1, w2, b2, c_r, w, w_r, a)


NB = 3


def _edge_body(s2_hbm, s3_hbm, t_hbm, src_hbm, dst_hbm, et_hbm, z_hbm, out_hbm,
               acc, *scr):
    bufs = []
    for b in range(NB):
        bufs.append(scr[b * 7:(b + 1) * 7])
    gsems = scr[NB * 7:NB * 7 + NB]
    ssems = scr[NB * 7 + NB:NB * 7 + 2 * NB]

    cid = lax.axis_index("c")
    sid = lax.axis_index("s")
    wid = cid * NS + sid
    ebase = wid * EPW

    def fire(b, j):
        src_v, dst_v, et_v, sidx_v, g1_v, s3_v, rows_v = bufs[b]
        off = ebase + j * C
        pltpu.sync_copy(src_hbm.at[pl.ds(off, C)], src_v)
        pltpu.sync_copy(dst_hbm.at[pl.ds(off, C)], dst_v)
        pltpu.sync_copy(et_hbm.at[pl.ds(off, C)], et_v)
        for k in range(C // 16):
            sl = pl.ds(k * 16, 16)
            sidx_v[sl] = src_v[sl] * RP + et_v[sl]
        pltpu.async_copy(s2_hbm.at[sidx_v], g1_v, gsems[b])
        pltpu.async_copy(s3_hbm.at[dst_v], s3_v, gsems[b])
        pltpu.async_copy(t_hbm.at[dst_v], rows_v, gsems[b])

    def drain_scatter(b):
        src_v, dst_v, et_v, sidx_v, g1_v, s3_v, rows_v = bufs[b]
        pltpu.make_async_copy(rows_v, acc.at[src_v], ssems[b]).wait()

    def compute_and_scatter(b):
        src_v, dst_v, et_v, sidx_v, g1_v, s3_v, rows_v = bufs[b]
        pltpu.make_async_copy(s2_hbm.at[sidx_v], g1_v, gsems[b]).wait()
        pltpu.make_async_copy(s3_hbm.at[dst_v], s3_v, gsems[b]).wait()
        pltpu.make_async_copy(t_hbm.at[dst_v], rows_v, gsems[b]).wait()

        def kbody(k, carry):
            sl = pl.ds(k * 16, 16)
            sc = g1_v[sl] + s3_v[sl]
            lr = jnp.where(sc > 0.0, sc, sc * ALPHA)
            ee = jnp.exp(-lr)
            for jl in range(16):
                eei = jnp.broadcast_to(lax.slice(ee, (jl,), (jl + 1,)), (16,))
                i = k * 16 + jl
                for cc in range(TW // 16):
                    slc = pl.ds(cc * 16, 16)
                    rows_v[i, slc] = rows_v[i, slc] * eei
            return carry

        lax.fori_loop(0, C // 16, kbody, 0)
        pltpu.async_copy(rows_v, acc.at[src_v], ssems[b], add=True)

    def slot(j, b, bn, guard_drain):
        if guard_drain:
            @pl.when(j - 2 >= 0)
            def _():
                drain_scatter(bn)
        else:
            drain_scatter(bn)
        fire(bn, j + 1)
        compute_and_scatter(b)

    fire(0, 0)
    pltpu.sync_copy(z_hbm.at[pl.ds(sid * RPT, RPT)],
                    acc.at[pl.ds(sid * RPT, RPT)])
    plsc.subcore_barrier()

    def main(jj, carry):
        j0 = jj * NB
        slot(j0 + 0, 0, 1, True)
        slot(j0 + 1, 1, 2, True)
        slot(j0 + 2, 2, 0, False)
        return carry

    lax.fori_loop(0, (NCHUNK - 2) // NB, main, 0)
    drain_scatter(1)
    fire(1, NCHUNK - 1)
    compute_and_scatter(0)
    compute_and_scatter(1)
    drain_scatter(2)
    drain_scatter(0)
    drain_scatter(1)
    plsc.subcore_barrier()
    pltpu.sync_copy(acc.at[pl.ds(sid * RPT, RPT)],
                    out_hbm.at[cid, pl.ds(sid * RPT, RPT)])


_edge_scratch = [pltpu.VMEM_SHARED((NPAD, TW), jnp.float32)]
for _b in range(NB):
    _edge_scratch += [
        pltpu.VMEM((C,), jnp.int32),
        pltpu.VMEM((C,), jnp.int32),
        pltpu.VMEM((C,), jnp.int32),
        pltpu.VMEM((C,), jnp.int32),
        pltpu.VMEM((C,), jnp.float32),
        pltpu.VMEM((C,), jnp.float32),
        pltpu.VMEM((C, TW), jnp.float32),
    ]
_edge_scratch += [pltpu.SemaphoreType.DMA] * (2 * NB)

_edge_kernel = functools.partial(
    pl.kernel,
    out_type=jax.ShapeDtypeStruct((NC, NPAD, TW), jnp.float32),
    mesh=plsc.VectorSubcoreMesh(core_axis_name="c", subcore_axis_name="s"),
    compiler_params=pltpu.CompilerParams(use_tc_tiling_on_sc=False),
    scratch_types=_edge_scratch,
)(_edge_body)


def _combine_body(p_ref, o_ref):
    h = p_ref[0, :, 0:D] + p_ref[1, :, 0:D]
    rs = p_ref[0, :, D:D + 1] + p_ref[1, :, D:D + 1]
    den = jnp.where(rs == 0.0, 1e-12, rs)
    o = h / den
    o_ref[...] = jnp.where(o > 0.0, o, jnp.exp(jnp.minimum(o, 0.0)) - 1.0)


def _combine(partial):
    bo = 1000
    return pl.pallas_call(
        _combine_body,
        grid=(N // bo,),
        in_specs=[pl.BlockSpec((NC, bo, TW), lambda i: (0, i, 0))],
        out_specs=pl.BlockSpec((bo, D), lambda i: (i, 0)),
        out_shape=jax.ShapeDtypeStruct((N, D), jnp.float32),
    )(partial)


def kernel(input, relation_embeds, edge_list, edge_type, W1, b1, W2, b2,
           c_r, W, W_r, a):
    relp = jnp.pad(relation_embeds, ((0, RP - relation_embeds.shape[0]), (0, 0)))
    s_tab, t_tab, s3_tab = _build_tables(
        input, relp, W1, b1.reshape(1, HID), W2, b2.reshape(1, D),
        c_r, W, W_r, a)
    s2 = s_tab.reshape(N * RP)
    s3 = s3_tab.reshape(N)
    src = edge_list[0]
    dst = edge_list[1]
    zeros = jnp.zeros((NPAD, TW), jnp.float32)
    partial = _edge_kernel(s2, s3, t_tab, src, dst, edge_type, zeros)
    return _combine(partial)


# speedup vs baseline: 14.8208x; 1.0694x over previous
import functools

import jax
import jax.numpy as jnp
from jax import lax
from jax.experimental import pallas as pl
from jax.experimental.pallas import tpu as pltpu
from jax.experimental.pallas import tpu_sc as plsc

N = 10000
E = 320000
D = 128
RP = 128
HID = 512
NC, NS = 2, 16
NW = NC * NS
EPW = E // NW
C = 80
NCHUNK = EPW // C
NPAD = 10240
RPT = NPAD // NS
ALPHA = 0.2
NB = 4


def _tables_body(x_ref, rel_ref, w1_ref, b1_ref, w2_ref, b2_ref, cr_ref,
                 w_ref, wr_ref, a_ref, s_ref, t_ref, s3_ref):
    x = x_ref[...]
    rel = rel_ref[...]
    h1 = jnp.maximum(
        jnp.dot(rel, w1_ref[...], preferred_element_type=jnp.float32)
        + b1_ref[...], 0.0)
    theta = (jnp.dot(h1, w2_ref[...], preferred_element_type=jnp.float32)
             + b2_ref[...])
    a1 = a_ref[:, 0:D]
    a2 = a_ref[:, D:2 * D]
    a3 = a_ref[:, 2 * D:3 * D]
    nt = (((1,), (1,)), ((), ()))
    v = lax.dot_general(a2, wr_ref[...], nt,
                        preferred_element_type=jnp.float32)
    c2 = jnp.sum(cr_ref[...] * v)
    q = jnp.cos(theta) * v
    wa1 = lax.dot_general(a1, w_ref[...], nt,
                          preferred_element_type=jnp.float32)
    wa3 = lax.dot_general(a3, w_ref[...], nt,
                          preferred_element_type=jnp.float32)
    spart = lax.dot_general(x, wa1, nt,
                            preferred_element_type=jnp.float32)
    s3 = lax.dot_general(x, wa3, nt,
                         preferred_element_type=jnp.float32)
    s_ref[...] = lax.dot_general(x, q, nt,
                                 preferred_element_type=jnp.float32) + spart + c2
    t_ref[...] = jnp.dot(x, w_ref[...], preferred_element_type=jnp.float32)
    s3_ref[...] = s3


def _build_tables(x, relp, w1, b1, w2, b2, c_r, w, w_r, a):
    bn = 2000
    return pl.pallas_call(
        _tables_body,
        grid=(N // bn,),
        in_specs=[
            pl.BlockSpec((bn, D), lambda i: (i, 0)),
            pl.BlockSpec((RP, D), lambda i: (0, 0)),
            pl.BlockSpec((D, HID), lambda i: (0, 0)),
            pl.BlockSpec((1, HID), lambda i: (0, 0)),
            pl.BlockSpec((HID, D), lambda i: (0, 0)),
            pl.BlockSpec((1, D), lambda i: (0, 0)),
            pl.BlockSpec((1, D), lambda i: (0, 0)),
            pl.BlockSpec((D, D), lambda i: (0, 0)),
            pl.BlockSpec((D, D), lambda i: (0, 0)),
            pl.BlockSpec((1, 3 * D), lambda i: (0, 0)),
        ],
        out_specs=[
            pl.BlockSpec((bn, RP), lambda i: (i, 0)),
            pl.BlockSpec((bn, D), lambda i: (i, 0)),
            pl.BlockSpec((bn, 1), lambda i: (i, 0)),
        ],
        out_shape=[
            jax.ShapeDtypeStruct((N, RP), jnp.float32),
            jax.ShapeDtypeStruct((N, D), jnp.float32),
            jax.ShapeDtypeStruct((N, 1), jnp.float32),
        ],
    )(x, relp, w1, b1, w2, b2, c_r, w, w_r, a)


def _edge_body(s2_hbm, s3_hbm, t_hbm, src_hbm, dst_hbm, et_hbm, z_hbm, z1_hbm,
               outh_hbm, outrs_hbm, acc, rs_acc, *scr):
    bufs = []
    for b in range(NB):
        bufs.append(scr[b * 8:(b + 1) * 8])
    gsems = scr[NB * 8:NB * 8 + NB]
    ssems = scr[NB * 8 + NB:NB * 8 + 2 * NB]

    cid = lax.axis_index("c")
    sid = lax.axis_index("s")
    wid = cid * NS + sid
    ebase = wid * EPW

    def fire(b, j):
        src_v, dst_v, et_v, sidx_v, g1_v, s3_v, ee_v, rows_v = bufs[b]
        off = ebase + j * C
        pltpu.sync_copy(src_hbm.at[pl.ds(off, C)], src_v)
        pltpu.sync_copy(dst_hbm.at[pl.ds(off, C)], dst_v)
        pltpu.sync_copy(et_hbm.at[pl.ds(off, C)], et_v)
        for k in range(C // 16):
            sl = pl.ds(k * 16, 16)
            sidx_v[sl] = src_v[sl] * RP + et_v[sl]
        pltpu.async_copy(s2_hbm.at[sidx_v], g1_v, gsems[b])
        pltpu.async_copy(s3_hbm.at[dst_v], s3_v, gsems[b])
        pltpu.async_copy(t_hbm.at[dst_v], rows_v, gsems[b])

    def drain_scatter(b):
        src_v, dst_v, et_v, sidx_v, g1_v, s3_v, ee_v, rows_v = bufs[b]
        pltpu.make_async_copy(rows_v, acc.at[src_v], ssems[b]).wait()
        pltpu.make_async_copy(ee_v, rs_acc.at[src_v], ssems[b]).wait()

    def compute_and_scatter(b):
        src_v, dst_v, et_v, sidx_v, g1_v, s3_v, ee_v, rows_v = bufs[b]
        pltpu.make_async_copy(s2_hbm.at[sidx_v], g1_v, gsems[b]).wait()
        pltpu.make_async_copy(s3_hbm.at[dst_v], s3_v, gsems[b]).wait()
        pltpu.make_async_copy(t_hbm.at[dst_v], rows_v, gsems[b]).wait()

        def kbody(k, carry):
            sl = pl.ds(k * 16, 16)
            sc = g1_v[sl] + s3_v[sl]
            lr = jnp.where(sc > 0.0, sc, sc * ALPHA)
            ee = jnp.exp(-lr)
            ee_v[sl] = ee
            for jl in range(16):
                eei = jnp.broadcast_to(lax.slice(ee, (jl,), (jl + 1,)), (16,))
                i = k * 16 + jl
                for cc in range(D // 16):
                    slc = pl.ds(cc * 16, 16)
                    rows_v[i, slc] = rows_v[i, slc] * eei
            return carry

        lax.fori_loop(0, C // 16, kbody, 0)
        pltpu.async_copy(rows_v, acc.at[src_v], ssems[b], add=True)
        pltpu.async_copy(ee_v, rs_acc.at[src_v], ssems[b], add=True)

    def slot(j, b, bn, guard_drain, guard_fire):
        def prefetch():
            if guard_drain:
                @pl.when(j - 2 >= 0)
                def _():
                    drain_scatter(bn)
            else:
                drain_scatter(bn)
            fire(bn, j + 2)

        if guard_fire:
            @pl.when(j + 2 <= NCHUNK - 1)
            def _():
                prefetch()
        else:
            prefetch()
        compute_and_scatter(b)

    fire(0, 0)
    fire(1, 1)
    pltpu.sync_copy(z_hbm.at[pl.ds(sid * RPT, RPT)],
                    acc.at[pl.ds(sid * RPT, RPT)])
    pltpu.sync_copy(z1_hbm.at[pl.ds(sid * RPT, RPT)],
                    rs_acc.at[pl.ds(sid * RPT, RPT)])
    plsc.subcore_barrier()

    def main(jj, carry):
        j0 = jj * NB
        slot(j0 + 0, 0, 2, True, False)
        slot(j0 + 1, 1, 3, True, False)
        slot(j0 + 2, 2, 0, False, False)
        slot(j0 + 3, 3, 1, False, True)
        return carry

    lax.fori_loop(0, (NCHUNK - 1) // NB, main, 0)
    compute_and_scatter((NCHUNK - 1) % NB)
    for j in range(NCHUNK - NB, NCHUNK):
        drain_scatter(j % NB)
    plsc.subcore_barrier()
    pltpu.sync_copy(acc.at[pl.ds(sid * RPT, RPT)],
                    outh_hbm.at[cid, pl.ds(sid * RPT, RPT)])
    pltpu.sync_copy(rs_acc.at[pl.ds(sid * RPT, RPT)],
                    outrs_hbm.at[cid, pl.ds(sid * RPT, RPT)])


_edge_scratch = [
    pltpu.VMEM_SHARED((NPAD, D), jnp.float32),
    pltpu.VMEM_SHARED((NPAD,), jnp.float32),
]
for _b in range(NB):
    _edge_scratch += [
        pltpu.VMEM((C,), jnp.int32),
        pltpu.VMEM((C,), jnp.int32),
        pltpu.VMEM((C,), jnp.int32),
        pltpu.VMEM((C,), jnp.int32),
        pltpu.VMEM((C,), jnp.float32),
        pltpu.VMEM((C,), jnp.float32),
        pltpu.VMEM((C,), jnp.float32),
        pltpu.VMEM((C, D), jnp.float32),
    ]
_edge_scratch += [pltpu.SemaphoreType.DMA] * (2 * NB)

_edge_kernel = functools.partial(
    pl.kernel,
    out_type=[
        jax.ShapeDtypeStruct((NC, NPAD, D), jnp.float32),
        jax.ShapeDtypeStruct((NC, NPAD), jnp.float32),
    ],
    mesh=plsc.VectorSubcoreMesh(core_axis_name="c", subcore_axis_name="s"),
    compiler_params=pltpu.CompilerParams(use_tc_tiling_on_sc=False),
    scratch_types=_edge_scratch,
)(_edge_body)


def _combine_body(p_ref, rs_ref, o_ref):
    h = p_ref[0] + p_ref[1]
    rs = rs_ref[0] + rs_ref[1]
    den = jnp.where(rs == 0.0, 1e-12, rs)
    o = h / den
    o_ref[...] = jnp.where(o > 0.0, o, jnp.exp(jnp.minimum(o, 0.0)) - 1.0)


def _combine(partial_h, partial_rs):
    bo = 1000
    return pl.pallas_call(
        _combine_body,
        grid=(N // bo,),
        in_specs=[
            pl.BlockSpec((NC, bo, D), lambda i: (0, i, 0)),
            pl.BlockSpec((NC, bo, 1), lambda i: (0, i, 0)),
        ],
        out_specs=pl.BlockSpec((bo, D), lambda i: (i, 0)),
        out_shape=jax.ShapeDtypeStruct((N, D), jnp.float32),
    )(partial_h, partial_rs)


def kernel(input, relation_embeds, edge_list, edge_type, W1, b1, W2, b2,
           c_r, W, W_r, a):
    relp = jnp.pad(relation_embeds, ((0, RP - relation_embeds.shape[0]), (0, 0)))
    s_tab, t_tab, s3_tab = _build_tables(
        input, relp, W1, b1.reshape(1, HID), W2, b2.reshape(1, D),
        c_r, W, W_r, a)
    s2 = s_tab.reshape(N * RP)
    s3 = s3_tab.reshape(N)
    src = edge_list[0]
    dst = edge_list[1]
    zeros = jnp.zeros((NPAD, D), jnp.float32)
    zeros1 = jnp.zeros((NPAD,), jnp.float32)
    partial_h, partial_rs = _edge_kernel(s2, s3, t_tab, src, dst, edge_type,
                                         zeros, zeros1)
    return _combine(partial_h, partial_rs.reshape(NC, NPAD, 1))


# speedup vs baseline: 24.3291x; 1.6416x over previous
import functools

import jax
import jax.numpy as jnp
from jax import lax
from jax.experimental import pallas as pl
from jax.experimental.pallas import tpu as pltpu
from jax.experimental.pallas import tpu_sc as plsc

N = 10000
E = 320000
D = 128
RP = 128
HID = 512
NC, NS = 2, 16
NW = NC * NS
EPW = E // NW
C = 80
NCHUNK = EPW // C
NPAD = 10240
RPT = NPAD // NS
ALPHA = 0.2
NB = 4


def _tables_body(x_ref, rel_ref, w1_ref, b1_ref, w2_ref, b2_ref, cr_ref,
                 w_ref, wr_ref, a_ref, src_ref, et_ref,
                 s_ref, t_ref, s3_ref, sidx_ref):
    @pl.when(pl.program_id(0) == 0)
    def _():
        sidx_ref[...] = src_ref[...] * RP + et_ref[...]
    x = x_ref[...]
    rel = rel_ref[...]
    h1 = jnp.maximum(
        jnp.dot(rel, w1_ref[...], preferred_element_type=jnp.float32)
        + b1_ref[...], 0.0)
    theta = (jnp.dot(h1, w2_ref[...], preferred_element_type=jnp.float32)
             + b2_ref[...])
    a1 = a_ref[:, 0:D]
    a2 = a_ref[:, D:2 * D]
    a3 = a_ref[:, 2 * D:3 * D]
    nt = (((1,), (1,)), ((), ()))
    v = lax.dot_general(a2, wr_ref[...], nt,
                        preferred_element_type=jnp.float32)
    c2 = jnp.sum(cr_ref[...] * v)
    q = jnp.cos(theta) * v
    wa1 = lax.dot_general(a1, w_ref[...], nt,
                          preferred_element_type=jnp.float32)
    wa3 = lax.dot_general(a3, w_ref[...], nt,
                          preferred_element_type=jnp.float32)
    spart = lax.dot_general(x, wa1, nt,
                            preferred_element_type=jnp.float32)
    s3 = lax.dot_general(x, wa3, nt,
                         preferred_element_type=jnp.float32)
    s_ref[...] = lax.dot_general(x, q, nt,
                                 preferred_element_type=jnp.float32) + spart + c2
    t_ref[...] = jnp.dot(x, w_ref[...], preferred_element_type=jnp.float32)
    s3_ref[...] = s3


def _build_tables(x, relp, w1, b1, w2, b2, c_r, w, w_r, a, src2d, et2d):
    bn = 2000
    be = E // D
    return pl.pallas_call(
        _tables_body,
        grid=(N // bn,),
        in_specs=[
            pl.BlockSpec((bn, D), lambda i: (i, 0)),
            pl.BlockSpec((RP, D), lambda i: (0, 0)),
            pl.BlockSpec((D, HID), lambda i: (0, 0)),
            pl.BlockSpec((1, HID), lambda i: (0, 0)),
            pl.BlockSpec((HID, D), lambda i: (0, 0)),
            pl.BlockSpec((1, D), lambda i: (0, 0)),
            pl.BlockSpec((1, D), lambda i: (0, 0)),
            pl.BlockSpec((D, D), lambda i: (0, 0)),
            pl.BlockSpec((D, D), lambda i: (0, 0)),
            pl.BlockSpec((1, 3 * D), lambda i: (0, 0)),
            pl.BlockSpec((be, D), lambda i: (0, 0)),
            pl.BlockSpec((be, D), lambda i: (0, 0)),
        ],
        out_specs=[
            pl.BlockSpec((bn, RP), lambda i: (i, 0)),
            pl.BlockSpec((bn, D), lambda i: (i, 0)),
            pl.BlockSpec((bn, 1), lambda i: (i, 0)),
            pl.BlockSpec((be, D), lambda i: (0, 0)),
        ],
        out_shape=[
            jax.ShapeDtypeStruct((N, RP), jnp.float32),
            jax.ShapeDtypeStruct((N, D), jnp.float32),
            jax.ShapeDtypeStruct((N, 1), jnp.float32),
            jax.ShapeDtypeStruct((E // D, D), jnp.int32),
        ],
    )(x, relp, w1, b1, w2, b2, c_r, w, w_r, a, src2d, et2d)


CPB = 8
HALF = 4 * C
NBODY = (NCHUNK - 5) // CPB


def _edge_body(s2_hbm, s3_hbm, t_hbm, src_hbm, dst_hbm, sidx_hbm, z_hbm,
               z1_hbm, outh_hbm, outrs_hbm, acc, rs_acc, *scr):
    bufs = []
    for b in range(NB):
        bufs.append(scr[b * 5:(b + 1) * 5])
    o = NB * 5
    pairs = [scr[o:o + 3], scr[o + 3:o + 6]]
    gsems = scr[o + 6:o + 6 + NB]
    ssems = scr[o + 6 + NB:o + 6 + 2 * NB]
    isems = scr[o + 6 + 2 * NB:o + 8 + 2 * NB]

    cid = lax.axis_index("c")
    sid = lax.axis_index("s")
    wid = cid * NS + sid
    ebase = wid * EPW

    def prefetch_idx(p, off):
        src_p, dst_p, sidx_p = pairs[p]
        pltpu.async_copy(src_hbm.at[pl.ds(off, HALF)], src_p, isems[p])
        pltpu.async_copy(dst_hbm.at[pl.ds(off, HALF)], dst_p, isems[p])
        pltpu.async_copy(sidx_hbm.at[pl.ds(off, HALF)], sidx_p, isems[p])

    def wait_idx(p, off):
        src_p, dst_p, sidx_p = pairs[p]
        pltpu.make_async_copy(src_hbm.at[pl.ds(off, HALF)], src_p,
                              isems[p]).wait()
        pltpu.make_async_copy(dst_hbm.at[pl.ds(off, HALF)], dst_p,
                              isems[p]).wait()
        pltpu.make_async_copy(sidx_hbm.at[pl.ds(off, HALF)], sidx_p,
                              isems[p]).wait()

    def fire(b, p, pos):
        src_v, g1_v, s3_v, ee_v, rows_v = bufs[b]
        src_p, dst_p, sidx_p = pairs[p]
        for k in range(C // 16):
            src_v[pl.ds(k * 16, 16)] = src_p[pl.ds(pos * C + k * 16, 16)]
        dsl = dst_p.at[pl.ds(pos * C, C)]
        pltpu.async_copy(s2_hbm.at[sidx_p.at[pl.ds(pos * C, C)]], g1_v,
                         gsems[b])
        pltpu.async_copy(s3_hbm.at[dsl], s3_v, gsems[b])
        pltpu.async_copy(t_hbm.at[dsl], rows_v, gsems[b])

    def drain_scatter(b):
        src_v, g1_v, s3_v, ee_v, rows_v = bufs[b]
        pltpu.make_async_copy(rows_v, acc.at[src_v], ssems[b]).wait()
        pltpu.make_async_copy(ee_v, rs_acc.at[src_v], ssems[b]).wait()

    def compute_and_scatter(b, p, pos):
        src_v, g1_v, s3_v, ee_v, rows_v = bufs[b]
        src_p, dst_p, sidx_p = pairs[p]
        dsl = dst_p.at[pl.ds(pos * C, C)]
        pltpu.make_async_copy(s2_hbm.at[sidx_p.at[pl.ds(pos * C, C)]], g1_v,
                              gsems[b]).wait()
        pltpu.make_async_copy(s3_hbm.at[dsl], s3_v, gsems[b]).wait()
        pltpu.make_async_copy(t_hbm.at[dsl], rows_v, gsems[b]).wait()

        def kbody(k, carry):
            sl = pl.ds(k * 16, 16)
            sc = g1_v[sl] + s3_v[sl]
            lr = jnp.where(sc > 0.0, sc, sc * ALPHA)
            ee = jnp.exp(-lr)
            ee_v[sl] = ee
            for jl in range(16):
                eei = jnp.broadcast_to(lax.slice(ee, (jl,), (jl + 1,)), (16,))
                i = k * 16 + jl
                for cc in range(D // 16):
                    slc = pl.ds(cc * 16, 16)
                    rows_v[i, slc] = rows_v[i, slc] * eei
            return carry

        lax.fori_loop(0, C // 16, kbody, 0)
        pltpu.async_copy(rows_v, acc.at[src_v], ssems[b], add=True)
        pltpu.async_copy(ee_v, rs_acc.at[src_v], ssems[b], add=True)


    def slot(j, s, guard_drain):
        b = s % 4
        bn = (s + 2) % 4
        fp, fpos = ((s + 2) // 4) % 2, (s + 2) % 4
        if guard_drain:
            @pl.when(j - 2 >= 0)
            def _():
                drain_scatter(bn)
        else:
            drain_scatter(bn)
        fire(bn, fp, fpos)
        compute_and_scatter(b, (s // 4) % 2, s % 4)

    prefetch_idx(0, ebase)
    wait_idx(0, ebase)
    prefetch_idx(1, ebase + HALF)
    fire(0, 0, 0)
    fire(1, 0, 1)
    pltpu.sync_copy(z_hbm.at[pl.ds(sid * RPT, RPT)],
                    acc.at[pl.ds(sid * RPT, RPT)])
    pltpu.sync_copy(z1_hbm.at[pl.ds(sid * RPT, RPT)],
                    rs_acc.at[pl.ds(sid * RPT, RPT)])
    plsc.subcore_barrier()

    def main(kk, carry):
        j0 = kk * CPB
        slot(j0 + 0, 0, True)
        slot(j0 + 1, 1, True)
        wait_idx(1, ebase + j0 * C + HALF)
        prefetch_idx(0, ebase + (j0 + CPB) * C)
        slot(j0 + 2, 2, False)
        slot(j0 + 3, 3, False)
        slot(j0 + 4, 4, False)
        slot(j0 + 5, 5, False)
        wait_idx(0, ebase + (j0 + CPB) * C)
        prefetch_idx(1, ebase + (j0 + CPB) * C + HALF)
        slot(j0 + 6, 6, False)
        slot(j0 + 7, 7, False)
        return carry

    lax.fori_loop(0, NBODY, main, 0)
    drain_scatter(2)
    fire(2, 0, 2)
    compute_and_scatter(0, 0, 0)
    drain_scatter(3)
    fire(3, 0, 3)
    compute_and_scatter(1, 0, 1)
    drain_scatter(0)
    wait_idx(1, ebase + (NCHUNK - 1) * C)
    fire(0, 1, 0)
    compute_and_scatter(2, 0, 2)
    compute_and_scatter(3, 0, 3)
    compute_and_scatter(0, 1, 0)
    drain_scatter(1)
    drain_scatter(2)
    drain_scatter(3)
    drain_scatter(0)
    plsc.subcore_barrier()
    pltpu.sync_copy(acc.at[pl.ds(sid * RPT, RPT)],
                    outh_hbm.at[cid, pl.ds(sid * RPT, RPT)])
    pltpu.sync_copy(rs_acc.at[pl.ds(sid * RPT, RPT)],
                    outrs_hbm.at[cid, pl.ds(sid * RPT, RPT)])


_edge_scratch = [
    pltpu.VMEM_SHARED((NPAD, D), jnp.float32),
    pltpu.VMEM_SHARED((NPAD,), jnp.float32),
]
for _b in range(NB):
    _edge_scratch += [
        pltpu.VMEM((C,), jnp.int32),
        pltpu.VMEM((C,), jnp.float32),
        pltpu.VMEM((C,), jnp.float32),
        pltpu.VMEM((C,), jnp.float32),
        pltpu.VMEM((C, D), jnp.float32),
    ]
for _p in range(2):
    _edge_scratch += [
        pltpu.VMEM((HALF,), jnp.int32),
        pltpu.VMEM((HALF,), jnp.int32),
        pltpu.VMEM((HALF,), jnp.int32),
    ]
_edge_scratch += [pltpu.SemaphoreType.DMA] * (2 * NB + 2)

_edge_kernel = functools.partial(
    pl.kernel,
    out_type=[
        jax.ShapeDtypeStruct((NC, NPAD, D), jnp.float32),
        jax.ShapeDtypeStruct((NC, NPAD), jnp.float32),
    ],
    mesh=plsc.VectorSubcoreMesh(core_axis_name="c", subcore_axis_name="s"),
    compiler_params=pltpu.CompilerParams(use_tc_tiling_on_sc=False),
    scratch_types=_edge_scratch,
)(_edge_body)


def _combine_body(p_ref, rs_ref, o_ref):
    h = p_ref[0] + p_ref[1]
    rs = rs_ref[0] + rs_ref[1]
    den = jnp.where(rs == 0.0, 1e-12, rs)
    o = h / den
    o_ref[...] = jnp.where(o > 0.0, o, jnp.exp(jnp.minimum(o, 0.0)) - 1.0)


def _combine(partial_h, partial_rs):
    bo = 1000
    return pl.pallas_call(
        _combine_body,
        grid=(N // bo,),
        in_specs=[
            pl.BlockSpec((NC, bo, D), lambda i: (0, i, 0)),
            pl.BlockSpec((NC, bo, 1), lambda i: (0, i, 0)),
        ],
        out_specs=pl.BlockSpec((bo, D), lambda i: (i, 0)),
        out_shape=jax.ShapeDtypeStruct((N, D), jnp.float32),
    )(partial_h, partial_rs)


def kernel(input, relation_embeds, edge_list, edge_type, W1, b1, W2, b2,
           c_r, W, W_r, a):
    relp = jnp.pad(relation_embeds, ((0, RP - relation_embeds.shape[0]), (0, 0)))
    src = edge_list[0]
    dst = edge_list[1]
    s_tab, t_tab, s3_tab, sidx2d = _build_tables(
        input, relp, W1, b1.reshape(1, HID), W2, b2.reshape(1, D),
        c_r, W, W_r, a, src.reshape(E // D, D), edge_type.reshape(E // D, D))
    s2 = s_tab.reshape(N * RP)
    s3 = s3_tab.reshape(N)
    sidx = sidx2d.reshape(E)
    zeros = jnp.zeros((NPAD, D), jnp.float32)
    zeros1 = jnp.zeros((NPAD,), jnp.float32)
    partial_h, partial_rs = _edge_kernel(s2, s3, t_tab, src, dst, sidx,
                                         zeros, zeros1)
    return _combine(partial_h, partial_rs.reshape(NC, NPAD, 1))
